# baseline (speedup 1.0000x reference)
import functools

import jax
import jax.numpy as jnp
from jax import lax
from jax.experimental import pallas as pl
from jax.experimental.pallas import tpu as pltpu
from jax.experimental.pallas import tpu_sc as plsc

_F, _V, _D, _B = 26, 100000, 16, 16384
_CHUNK = 1024


def _sc_plane_gather(embT, linP, xT):
    mesh = plsc.VectorSubcoreMesh(core_axis_name="c", subcore_axis_name="s")

    @functools.partial(
        pl.kernel,
        out_type=(
            jax.ShapeDtypeStruct((52, 8, _B), jnp.float32),
            jax.ShapeDtypeStruct((4, 8, _B), jnp.float32),
        ),
        mesh=mesh,
        scratch_types=[
            pltpu.VMEM((_V,), jnp.float32),
            pltpu.VMEM((_CHUNK,), jnp.int32),
            pltpu.VMEM((_CHUNK,), jnp.int32),
            pltpu.VMEM((_CHUNK,), jnp.float32),
            pltpu.VMEM((_CHUNK,), jnp.float32),
            pltpu.SemaphoreType.DMA,
            pltpu.SemaphoreType.DMA,
            pltpu.SemaphoreType.DMA,
            pltpu.SemaphoreType.DMA,
        ],
        compiler_params=pltpu.CompilerParams(
            use_tc_tiling_on_sc=True, needs_layout_passes=False),
    )
    def k(embT_hbm, lin_hbm, xT_hbm, oe_hbm, ol_hbm, plane_v, idx_a, idx_b,
          out_a, out_b, si0, si1, so0, so1):
        cid = lax.axis_index("c")
        sid = lax.axis_index("s")
        half = sid // 8
        lane8 = sid % 8
        nch = _B // _CHUNK
        isem = (si0, si1)
        osem = (so0, so1)
        ibuf = (idx_a, idx_b)
        obuf = (out_a, out_b)

        def gather_plane(f, out_row, plane_src):
            pltpu.async_copy(xT_hbm.at[f, pl.ds(0, _CHUNK)], idx_a, si0)
            pltpu.sync_copy(plane_src, plane_v)
            for h in range(nch):
                p = h % 2
                ip = ibuf[p]
                op = obuf[p]
                pltpu.make_async_copy(
                    xT_hbm.at[f, pl.ds(h * _CHUNK, _CHUNK)], ip,
                    isem[p]).wait()
                if h + 1 < nch:
                    pn = (h + 1) % 2
                    pltpu.async_copy(
                        xT_hbm.at[f, pl.ds((h + 1) * _CHUNK, _CHUNK)],
                        ibuf[pn], isem[pn])
                if h >= 2:
                    pltpu.make_async_copy(
                        op, out_row.at[pl.ds((h - 2) * _CHUNK, _CHUNK)],
                        osem[p]).wait()

                @pl.loop(0, _CHUNK // 16)
                def _(i):
                    vals = plsc.load_gather(
                        plane_v, [ip[pl.ds(i * 16, 16)]])
                    op[pl.ds(i * 16, 16)] = vals

                pltpu.async_copy(
                    op, out_row.at[pl.ds(h * _CHUNK, _CHUNK)], osem[p])
            for h in (nch - 2, nch - 1):
                p = h % 2
                pltpu.make_async_copy(
                    obuf[p], out_row.at[pl.ds(h * _CHUNK, _CHUNK)],
                    osem[p]).wait()

        @pl.loop(0, 13)
        def _(r):
            group = cid * 26 + r * 2 + half
            p = group * 8 + lane8
            f = p // _D
            dd = p % _D
            gather_plane(f, oe_hbm.at[group, lane8], embT_hbm.at[f, dd])

        group = cid * 2 + half
        f = group * 8 + lane8

        @pl.when(f < _F)
        def _():
            gather_plane(f, ol_hbm.at[group, lane8], lin_hbm.at[f])

        @pl.when(f >= _F)
        def _():
            @pl.loop(0, _CHUNK // 16)
            def _(i):
                out_a[pl.ds(i * 16, 16)] = jnp.zeros((16,), jnp.float32)

            @pl.loop(0, _B // _CHUNK)
            def _(h):
                pltpu.sync_copy(
                    out_a,
                    ol_hbm.at[group, lane8, pl.ds(h * _CHUNK, _CHUNK)])

    return k(embT, linP, xT)


def _dense_body(xe_ref, lt_ref, w1t_ref, b1_ref, w2t_ref, b2_ref, w3t_ref,
                bb_ref, st_ref, o_ref):
    xeT = xe_ref[...]
    lin_sum = jnp.sum(lt_ref[...], axis=0, keepdims=True)
    sT = jnp.dot(st_ref[...], xeT, preferred_element_type=jnp.float32)
    fm = 0.5 * (jnp.sum(sT * sT, axis=0, keepdims=True)
                - jnp.sum(xeT * xeT, axis=0, keepdims=True))
    h = jnp.dot(w1t_ref[...], xeT, preferred_element_type=jnp.float32)
    h = jnp.maximum(h + b1_ref[...], 0.0)
    h = jnp.dot(w2t_ref[...], h, preferred_element_type=jnp.float32)
    h = jnp.maximum(h + b2_ref[...], 0.0)
    deep = jnp.dot(w3t_ref[...], h, preferred_element_type=jnp.float32)
    o_ref[...] = jax.nn.sigmoid(lin_sum + fm + deep + bb_ref[...])


def _dense(xeT, linT, w1t, b1c, w2t, b2c, w3t, bb, sT, bs):
    grid = (_B // bs,)
    full = lambda a: pl.BlockSpec(a.shape, lambda i: (0, 0))
    return pl.pallas_call(
        _dense_body,
        grid=grid,
        in_specs=[
            pl.BlockSpec((416, bs), lambda i: (0, i)),
            pl.BlockSpec((32, bs), lambda i: (0, i)),
            full(w1t), full(b1c), full(w2t), full(b2c), full(w3t),
            full(bb), full(sT),
        ],
        out_specs=pl.BlockSpec((1, bs), lambda i: (0, i)),
        out_shape=jax.ShapeDtypeStruct((1, _B), jnp.float32),
    )(xeT, linT, w1t, b1c, w2t, b2c, w3t, bb, sT)


def kernel(x, emb_tables, lin_tables, linear_bias, W1, b1, W2, b2, W3, b3):
    embT = jnp.transpose(emb_tables, (0, 2, 1))
    linP = jnp.transpose(lin_tables, (0, 2, 1))[:, 0, :]
    xT = x.T

    xeT4, linT4 = _sc_plane_gather(embT, linP, xT)
    xeT = xeT4.reshape(416, _B)
    linT = linT4.reshape(32, _B)

    sT = jnp.tile(jnp.eye(_D, dtype=jnp.float32), (1, _F))
    bb = (b3 + linear_bias).reshape(1, 1)
    out = _dense(xeT, linT, W1.T, b1.reshape(-1, 1), W2.T, b2.reshape(-1, 1),
                 W3.T, bb, sT, bs=2048)
    return out.reshape(_B, 1)


# speedup vs baseline: 1.2343x; 1.2343x over previous
import functools

import jax
import jax.numpy as jnp
from jax import lax
from jax.experimental import pallas as pl
from jax.experimental.pallas import tpu as pltpu
from jax.experimental.pallas import tpu_sc as plsc

_F, _V, _D, _B = 26, 100000, 16, 16384
_CHUNK = 2048


def _sc_plane_gather(embT, linP, xT):
    mesh = plsc.VectorSubcoreMesh(core_axis_name="c", subcore_axis_name="s")

    @functools.partial(
        pl.kernel,
        out_type=(
            jax.ShapeDtypeStruct((52, 8, _B), jnp.float32),
            jax.ShapeDtypeStruct((4, 8, _B), jnp.float32),
        ),
        mesh=mesh,
        scratch_types=[
            pltpu.VMEM((_V,), jnp.float32),
            pltpu.VMEM((_CHUNK,), jnp.int32),
            pltpu.VMEM((_CHUNK,), jnp.int32),
            pltpu.VMEM((_CHUNK,), jnp.float32),
            pltpu.VMEM((_CHUNK,), jnp.float32),
            pltpu.SemaphoreType.DMA,
            pltpu.SemaphoreType.DMA,
            pltpu.SemaphoreType.DMA,
            pltpu.SemaphoreType.DMA,
        ],
        compiler_params=pltpu.CompilerParams(
            use_tc_tiling_on_sc=True, needs_layout_passes=False),
    )
    def k(embT_hbm, lin_hbm, xT_hbm, oe_hbm, ol_hbm, plane_v, idx_a, idx_b,
          out_a, out_b, si0, si1, so0, so1):
        cid = lax.axis_index("c")
        sid = lax.axis_index("s")
        half = sid // 8
        lane8 = sid % 8
        nch = _B // _CHUNK
        isem = (si0, si1)
        osem = (so0, so1)
        ibuf = (idx_a, idx_b)
        obuf = (out_a, out_b)

        def gather_plane(f, out_row, plane_src):
            pltpu.async_copy(xT_hbm.at[f, pl.ds(0, _CHUNK)], idx_a, si0)
            pltpu.sync_copy(plane_src, plane_v)
            for h in range(nch):
                p = h % 2
                ip = ibuf[p]
                op = obuf[p]
                pltpu.make_async_copy(
                    xT_hbm.at[f, pl.ds(h * _CHUNK, _CHUNK)], ip,
                    isem[p]).wait()
                if h + 1 < nch:
                    pn = (h + 1) % 2
                    pltpu.async_copy(
                        xT_hbm.at[f, pl.ds((h + 1) * _CHUNK, _CHUNK)],
                        ibuf[pn], isem[pn])
                if h >= 2:
                    pltpu.make_async_copy(
                        op, out_row.at[pl.ds((h - 2) * _CHUNK, _CHUNK)],
                        osem[p]).wait()

                @pl.loop(0, _CHUNK // 16)
                def _(i):
                    vals = plsc.load_gather(
                        plane_v, [ip[pl.ds(i * 16, 16)]])
                    op[pl.ds(i * 16, 16)] = vals

                pltpu.async_copy(
                    op, out_row.at[pl.ds(h * _CHUNK, _CHUNK)], osem[p])
            for h in (nch - 2, nch - 1):
                p = h % 2
                pltpu.make_async_copy(
                    obuf[p], out_row.at[pl.ds(h * _CHUNK, _CHUNK)],
                    osem[p]).wait()

        @pl.loop(0, 13)
        def _(r):
            group = cid * 26 + r * 2 + half
            p = group * 8 + lane8
            f = p // _D
            dd = p % _D
            gather_plane(f, oe_hbm.at[group, lane8], embT_hbm.at[f, dd])

        group = cid * 2 + half
        f = group * 8 + lane8

        @pl.when(f < _F)
        def _():
            gather_plane(f, ol_hbm.at[group, lane8], lin_hbm.at[f])

        @pl.when(f >= _F)
        def _():
            @pl.loop(0, _CHUNK // 16)
            def _(i):
                out_a[pl.ds(i * 16, 16)] = jnp.zeros((16,), jnp.float32)

            @pl.loop(0, _B // _CHUNK)
            def _(h):
                pltpu.sync_copy(
                    out_a,
                    ol_hbm.at[group, lane8, pl.ds(h * _CHUNK, _CHUNK)])

    return k(embT, linP, xT)


def _dense_body(xe_ref, lt_ref, w1t_ref, b1_ref, w2t_ref, b2_ref, w3t_ref,
                bb_ref, st_ref, o_ref):
    xeT = xe_ref[...]
    lin_sum = jnp.sum(lt_ref[...], axis=0, keepdims=True)
    sT = jnp.dot(st_ref[...], xeT, preferred_element_type=jnp.float32)
    fm = 0.5 * (jnp.sum(sT * sT, axis=0, keepdims=True)
                - jnp.sum(xeT * xeT, axis=0, keepdims=True))
    h = jnp.dot(w1t_ref[...], xeT, preferred_element_type=jnp.float32)
    h = jnp.maximum(h + b1_ref[...], 0.0)
    h = jnp.dot(w2t_ref[...], h, preferred_element_type=jnp.float32)
    h = jnp.maximum(h + b2_ref[...], 0.0)
    deep = jnp.dot(w3t_ref[...], h, preferred_element_type=jnp.float32)
    o_ref[...] = jax.nn.sigmoid(lin_sum + fm + deep + bb_ref[...])


def _dense(xeT, linT, w1t, b1c, w2t, b2c, w3t, bb, sT, bs):
    grid = (_B // bs,)
    full = lambda a: pl.BlockSpec(a.shape, lambda i: (0, 0))
    return pl.pallas_call(
        _dense_body,
        grid=grid,
        in_specs=[
            pl.BlockSpec((416, bs), lambda i: (0, i)),
            pl.BlockSpec((32, bs), lambda i: (0, i)),
            full(w1t), full(b1c), full(w2t), full(b2c), full(w3t),
            full(bb), full(sT),
        ],
        out_specs=pl.BlockSpec((1, bs), lambda i: (0, i)),
        out_shape=jax.ShapeDtypeStruct((1, _B), jnp.float32),
    )(xeT, linT, w1t, b1c, w2t, b2c, w3t, bb, sT)


def kernel(x, emb_tables, lin_tables, linear_bias, W1, b1, W2, b2, W3, b3):
    embT = jnp.transpose(emb_tables, (0, 2, 1))
    linP = jnp.transpose(lin_tables, (0, 2, 1))[:, 0, :]
    xT = x.T

    xeT4, linT4 = _sc_plane_gather(embT, linP, xT)
    xeT = xeT4.reshape(416, _B)
    linT = linT4.reshape(32, _B)

    sT = jnp.tile(jnp.eye(_D, dtype=jnp.float32), (1, _F))
    bb = (b3 + linear_bias).reshape(1, 1)
    out = _dense(xeT, linT, W1.T, b1.reshape(-1, 1), W2.T, b2.reshape(-1, 1),
                 W3.T, bb, sT, bs=4096)
    return out.reshape(_B, 1)
